# Initial kernel scaffold; baseline (speedup 1.0000x reference)
import jax
import jax.numpy as jnp
from jax.experimental import pallas as pl


def kernel(x, edge_index, fc_w, fc_b, au_w, au_b, av_w, gl_w, gl_b, chaos_factor, ff1_w, ff1_b, ff2_w, ff2_b):
    raise NotImplementedError("write your pallas kernel here")


# speedup vs baseline: 6.9394x; 6.9394x over previous
import functools

import jax
import jax.numpy as jnp
from jax.experimental import pallas as pl
from jax.experimental.pallas import tpu as pltpu

N = 10000
E = 320000
DIM = 128
H = 8
HID = 2 * DIM

_BLK = 1000


def _proj_body(x_ref, fcw_ref, fcb_ref, w24_ref, b24_ref, chaos_ref,
               xfc_ref, tsl_ref, tsv_ref):
    xfc = jnp.dot(x_ref[...], fcw_ref[...],
                  preferred_element_type=jnp.float32) + fcb_ref[...]
    xfc_ref[...] = xfc
    hv = jnp.dot(xfc, w24_ref[...],
                 preferred_element_type=jnp.float32) + b24_ref[...]
    su = hv[:, 0:8]
    sv = hv[:, 8:16]
    lap = hv[:, 16:24] + chaos_ref[...]
    zeros = jnp.zeros_like(su)
    tsl_ref[...] = jnp.concatenate([su, su, zeros, lap], axis=1)
    tsv_ref[...] = jnp.concatenate([sv, sv], axis=1)


def _project(x, fc_w, fc_b, w24, b24, chaos):
    grid = (N // _BLK,)
    return pl.pallas_call(
        _proj_body,
        grid=grid,
        in_specs=[
            pl.BlockSpec((_BLK, DIM), lambda i: (i, 0)),
            pl.BlockSpec((DIM, DIM), lambda i: (0, 0)),
            pl.BlockSpec((DIM,), lambda i: (0,)),
            pl.BlockSpec((DIM, 24), lambda i: (0, 0)),
            pl.BlockSpec((24,), lambda i: (0,)),
            pl.BlockSpec((_BLK, H), lambda i: (i, 0)),
        ],
        out_specs=[
            pl.BlockSpec((_BLK, DIM), lambda i: (i, 0)),
            pl.BlockSpec((_BLK, 32), lambda i: (i, 0)),
            pl.BlockSpec((_BLK, 16), lambda i: (i, 0)),
        ],
        out_shape=[
            jax.ShapeDtypeStruct((N, DIM), jnp.float32),
            jax.ShapeDtypeStruct((N, 32), jnp.float32),
            jax.ShapeDtypeStruct((N, 16), jnp.float32),
        ],
    )(x, fc_w, fc_b, w24, b24, chaos)


def _ffn_body(xfc_ref, msg_ref, w1a_ref, w1b_ref, b1_ref, w2_ref, b2_ref,
              out_ref):
    h = (jnp.dot(xfc_ref[...], w1a_ref[...], preferred_element_type=jnp.float32)
         + jnp.dot(msg_ref[...], w1b_ref[...], preferred_element_type=jnp.float32)
         + b1_ref[...])
    h = 0.5 * h * (1.0 + jax.lax.erf(h * 0.7071067811865476))
    out_ref[...] = jnp.dot(h, w2_ref[...],
                           preferred_element_type=jnp.float32) + b2_ref[...]


def _ffn(xfc, msg, ff1_w, ff1_b, ff2_w, ff2_b):
    w1a = ff1_w[:DIM]
    w1b = ff1_w[DIM:]
    grid = (N // _BLK,)
    return pl.pallas_call(
        _ffn_body,
        grid=grid,
        in_specs=[
            pl.BlockSpec((_BLK, DIM), lambda i: (i, 0)),
            pl.BlockSpec((_BLK, DIM), lambda i: (i, 0)),
            pl.BlockSpec((DIM, HID), lambda i: (0, 0)),
            pl.BlockSpec((DIM, HID), lambda i: (0, 0)),
            pl.BlockSpec((HID,), lambda i: (0,)),
            pl.BlockSpec((HID, DIM), lambda i: (0, 0)),
            pl.BlockSpec((DIM,), lambda i: (0,)),
        ],
        out_specs=pl.BlockSpec((_BLK, DIM), lambda i: (i, 0)),
        out_shape=jax.ShapeDtypeStruct((N, DIM), jnp.float32),
    )(xfc, msg, w1a, w1b, ff1_b, ff2_w, ff2_b)


def kernel(x, edge_index, fc_w, fc_b, au_w, au_b, av_w, gl_w, gl_b,
           chaos_factor, ff1_w, ff1_b, ff2_w, ff2_b):
    src = edge_index[0]
    dst = edge_index[1]

    w24 = jnp.concatenate([au_w, av_w, gl_w], axis=1)
    b24 = jnp.concatenate([au_b, jnp.zeros((H,), jnp.float32), gl_b])
    chaos = (jax.random.normal(jax.random.key(42), (N, H), dtype=jnp.float32)
             * chaos_factor)

    xfc, tsl, tsv = _project(x, fc_w, fc_b, w24, b24, chaos)
    su = tsl[:, 0:8]
    sv = tsv[:, 0:8]
    laplace = tsl[:, 24:32]

    scores = su[src] + sv[dst]
    scores = jax.nn.leaky_relu(scores, negative_slope=0.2)
    unnorm = jnp.exp(scores)
    denom = jax.ops.segment_sum(unnorm, dst, num_segments=N)

    nbr_sum = jax.ops.segment_sum(laplace[src], dst, num_segments=N)
    indeg = jax.ops.segment_sum(jnp.ones((E,), jnp.float32), dst,
                                num_segments=N)
    nbr_avg = nbr_sum / jnp.maximum(indeg, 1.0)[:, None]
    energy = laplace - nbr_avg

    attn = (unnorm / denom[dst]) * jax.nn.sigmoid(energy[dst] - energy[src])
    msg = jax.ops.segment_sum(
        xfc[src] * jnp.tile(attn, (1, DIM // H)), dst, num_segments=N)

    return _ffn(xfc, msg, ff1_w, ff1_b, ff2_w, ff2_b)


# speedup vs baseline: 30.8018x; 4.4387x over previous
import functools

import jax
import jax.numpy as jnp
from jax import lax
from jax.experimental import pallas as pl
from jax.experimental.pallas import tpu as pltpu
from jax.experimental.pallas import tpu_sc as plsc

N = 10000
E = 320000
DIM = 128
H = 8
HID = 2 * DIM

_BLK = 1000
_NC, _NS = 2, 16
_CHUNK = 80
_ROWS = E // _CHUNK
_CPT = _ROWS // (_NC * _NS)
_NPAD = 10240
_STRIPE = _NPAD // _NS


def _proj_body(x_ref, fcw_ref, fcb_ref, w24_ref, b24_ref, chaos_ref,
               xfc_ref, tsl_ref, tsv_ref):
    xfc = jnp.dot(x_ref[...], fcw_ref[...],
                  preferred_element_type=jnp.float32) + fcb_ref[...]
    xfc_ref[...] = xfc
    hv = jnp.dot(xfc, w24_ref[...],
                 preferred_element_type=jnp.float32) + b24_ref[...]
    su = hv[:, 0:8]
    sv = hv[:, 8:16]
    lap = hv[:, 16:24] + chaos_ref[...]
    zeros = jnp.zeros_like(su)
    tsl_ref[...] = jnp.concatenate([su, su, zeros, lap], axis=1)
    tsv_ref[...] = jnp.concatenate([sv, sv], axis=1)


def _project(x, fc_w, fc_b, w24, b24, chaos):
    return pl.pallas_call(
        _proj_body,
        grid=(N // _BLK,),
        in_specs=[
            pl.BlockSpec((_BLK, DIM), lambda i: (i, 0)),
            pl.BlockSpec((DIM, DIM), lambda i: (0, 0)),
            pl.BlockSpec((DIM,), lambda i: (0,)),
            pl.BlockSpec((DIM, 24), lambda i: (0, 0)),
            pl.BlockSpec((24,), lambda i: (0,)),
            pl.BlockSpec((_BLK, H), lambda i: (i, 0)),
        ],
        out_specs=[
            pl.BlockSpec((_BLK, DIM), lambda i: (i, 0)),
            pl.BlockSpec((_BLK, 32), lambda i: (i, 0)),
            pl.BlockSpec((_BLK, 16), lambda i: (i, 0)),
        ],
        out_shape=[
            jax.ShapeDtypeStruct((N, DIM), jnp.float32),
            jax.ShapeDtypeStruct((N, 32), jnp.float32),
            jax.ShapeDtypeStruct((N, 16), jnp.float32),
        ],
    )(x, fc_w, fc_b, w24, b24, chaos)


def _sc_pass1(tsl, tsv, src2d, dst2d, z32):
    mesh = plsc.VectorSubcoreMesh(core_axis_name="c", subcore_axis_name="s")

    @functools.partial(
        pl.kernel,
        mesh=mesh,
        compiler_params=pltpu.CompilerParams(use_tc_tiling_on_sc=False),
        out_type=[
            jax.ShapeDtypeStruct((_NC, _NPAD, 32), jnp.float32),
            jax.ShapeDtypeStruct((E, 16), jnp.float32),
        ],
        scratch_types=[
            pltpu.VMEM((_CHUNK,), jnp.int32),
            pltpu.VMEM((_CHUNK,), jnp.int32),
            pltpu.VMEM((_CHUNK, 32), jnp.float32),
            pltpu.VMEM((_CHUNK, 16), jnp.float32),
            pltpu.VMEM((_CHUNK, 32), jnp.float32),
            pltpu.VMEM((_CHUNK, 16), jnp.float32),
            pltpu.VMEM_SHARED((_NPAD, 32), jnp.float32),
        ],
    )
    def kern(tsl_hbm, tsv_hbm, src_hbm, dst_hbm, z32_hbm, accp_hbm, pb_hbm,
             sidx, didx, tslg, tsvg, sb, pbuf, acc_sh):
        c = lax.axis_index("c")
        s = lax.axis_index("s")
        tile = c * _NS + s
        pltpu.sync_copy(z32_hbm, acc_sh.at[pl.ds(s * _STRIPE, _STRIPE)])
        lane = lax.iota(jnp.int32, 16)
        one0 = jnp.where(lane < 1, 1.0, 0.0).astype(jnp.float32)
        mask8 = lane < 8

        @pl.loop(0, _CHUNK)
        def _(r):
            sb[r, pl.ds(16, 16)] = one0

        plsc.subcore_barrier()

        @pl.loop(0, _CPT)
        def _(j):
            row = tile * _CPT + j
            pltpu.sync_copy(src_hbm.at[pl.ds(row * _CHUNK, _CHUNK)], sidx)
            pltpu.sync_copy(dst_hbm.at[pl.ds(row * _CHUNK, _CHUNK)], didx)
            pltpu.sync_copy(tsl_hbm.at[sidx], tslg)
            pltpu.sync_copy(tsv_hbm.at[didx], tsvg)

            @pl.loop(0, _CHUNK)
            def _(r):
                a0 = tslg[r, pl.ds(0, 16)]
                a1 = tslg[r, pl.ds(16, 16)]
                b0 = tsvg[r, pl.ds(0, 16)]
                t = a0 + b0
                t = jnp.maximum(t, 0.2 * t)
                p = jnp.exp(t)
                sb[r, pl.ds(0, 16)] = jnp.where(mask8, p, a1)
                pbuf[r, pl.ds(0, 16)] = p

            pltpu.sync_copy(sb, acc_sh.at[didx], add=True)
            pltpu.sync_copy(pbuf, pb_hbm.at[pl.ds(row * _CHUNK, _CHUNK)])

        plsc.subcore_barrier()
        pltpu.sync_copy(acc_sh.at[pl.ds(s * _STRIPE, _STRIPE)],
                        accp_hbm.at[c].at[pl.ds(s * _STRIPE, _STRIPE)])

    return kern(tsl, tsv, src2d, dst2d, z32)


def _energy_body(a0_ref, a1_ref, tsl_ref, tde_ref):
    acc = a0_ref[...] + a1_ref[...]
    denom = acc[:, 0:8]
    nbr = acc[:, 8:16]
    indeg = acc[:, 16:17]
    lap = tsl_ref[:, 24:32]
    energy = lap - nbr / jnp.maximum(indeg, 1.0)
    tde_ref[...] = jnp.concatenate([denom, denom, energy, energy], axis=1)


def _energy(accp, tsl):
    return pl.pallas_call(
        _energy_body,
        grid=(N // _BLK,),
        in_specs=[
            pl.BlockSpec((_BLK, 32), lambda i: (i, 0)),
            pl.BlockSpec((_BLK, 32), lambda i: (i, 0)),
            pl.BlockSpec((_BLK, 32), lambda i: (i, 0)),
        ],
        out_specs=pl.BlockSpec((_BLK, 32), lambda i: (i, 0)),
        out_shape=jax.ShapeDtypeStruct((_NPAD, 32), jnp.float32),
    )(accp[0], accp[1], tsl)


def _sc_pass2(tde, xfc, pb, src2d, dst2d, z128):
    mesh = plsc.VectorSubcoreMesh(core_axis_name="c", subcore_axis_name="s")

    @functools.partial(
        pl.kernel,
        mesh=mesh,
        compiler_params=pltpu.CompilerParams(use_tc_tiling_on_sc=False),
        out_type=jax.ShapeDtypeStruct((_NC, _NPAD, DIM), jnp.float32),
        scratch_types=[
            pltpu.VMEM((_CHUNK,), jnp.int32),
            pltpu.VMEM((_CHUNK,), jnp.int32),
            pltpu.VMEM((_CHUNK, 32), jnp.float32),
            pltpu.VMEM((_CHUNK, 32), jnp.float32),
            pltpu.VMEM((_CHUNK, DIM), jnp.float32),
            pltpu.VMEM((_CHUNK, 16), jnp.float32),
            pltpu.VMEM_SHARED((_NPAD, DIM), jnp.float32),
        ],
    )
    def kern(tde_hbm, xfc_hbm, pb_hbm, src_hbm, dst_hbm, z128_hbm, msgp_hbm,
             sidx, didx, gd, gs, xb, pbuf, msg_sh):
        c = lax.axis_index("c")
        s = lax.axis_index("s")
        tile = c * _NS + s
        pltpu.sync_copy(z128_hbm, msg_sh.at[pl.ds(s * _STRIPE, _STRIPE)])
        plsc.subcore_barrier()

        @pl.loop(0, _CPT)
        def _(j):
            row = tile * _CPT + j
            pltpu.sync_copy(src_hbm.at[pl.ds(row * _CHUNK, _CHUNK)], sidx)
            pltpu.sync_copy(dst_hbm.at[pl.ds(row * _CHUNK, _CHUNK)], didx)
            pltpu.sync_copy(tde_hbm.at[didx], gd)
            pltpu.sync_copy(tde_hbm.at[sidx], gs)
            pltpu.sync_copy(xfc_hbm.at[sidx], xb)
            pltpu.sync_copy(pb_hbm.at[pl.ds(row * _CHUNK, _CHUNK)], pbuf)

            @pl.loop(0, _CHUNK)
            def _(r):
                z = gd[r, pl.ds(16, 16)] - gs[r, pl.ds(16, 16)]
                sig = 1.0 / (1.0 + jnp.exp(-z))
                q = pbuf[r, pl.ds(0, 16)] / gd[r, pl.ds(0, 16)]
                m = q * sig
                for k in range(DIM // 16):
                    xb[r, pl.ds(k * 16, 16)] = xb[r, pl.ds(k * 16, 16)] * m

            pltpu.sync_copy(xb, msg_sh.at[didx], add=True)

        plsc.subcore_barrier()
        pltpu.sync_copy(msg_sh.at[pl.ds(s * _STRIPE, _STRIPE)],
                        msgp_hbm.at[c].at[pl.ds(s * _STRIPE, _STRIPE)])

    return kern(tde, xfc, pb, src2d, dst2d, z128)


def _ffn_body(xfc_ref, m0_ref, m1_ref, w1a_ref, w1b_ref, b1_ref, w2_ref,
              b2_ref, out_ref):
    msg = m0_ref[...] + m1_ref[...]
    h = (jnp.dot(xfc_ref[...], w1a_ref[...], preferred_element_type=jnp.float32)
         + jnp.dot(msg, w1b_ref[...], preferred_element_type=jnp.float32)
         + b1_ref[...])
    h = 0.5 * h * (1.0 + jax.lax.erf(h * 0.7071067811865476))
    out_ref[...] = jnp.dot(h, w2_ref[...],
                           preferred_element_type=jnp.float32) + b2_ref[...]


def _ffn(xfc, m0, m1, ff1_w, ff1_b, ff2_w, ff2_b):
    return pl.pallas_call(
        _ffn_body,
        grid=(N // _BLK,),
        in_specs=[
            pl.BlockSpec((_BLK, DIM), lambda i: (i, 0)),
            pl.BlockSpec((_BLK, DIM), lambda i: (i, 0)),
            pl.BlockSpec((_BLK, DIM), lambda i: (i, 0)),
            pl.BlockSpec((DIM, HID), lambda i: (0, 0)),
            pl.BlockSpec((DIM, HID), lambda i: (0, 0)),
            pl.BlockSpec((HID,), lambda i: (0,)),
            pl.BlockSpec((HID, DIM), lambda i: (0, 0)),
            pl.BlockSpec((DIM,), lambda i: (0,)),
        ],
        out_specs=pl.BlockSpec((_BLK, DIM), lambda i: (i, 0)),
        out_shape=jax.ShapeDtypeStruct((N, DIM), jnp.float32),
    )(xfc, m0, m1, ff1_w[:DIM], ff1_w[DIM:], ff1_b, ff2_w, ff2_b)


def kernel(x, edge_index, fc_w, fc_b, au_w, au_b, av_w, gl_w, gl_b,
           chaos_factor, ff1_w, ff1_b, ff2_w, ff2_b):
    src1d = edge_index[0]
    dst1d = edge_index[1]

    w24 = jnp.concatenate([au_w, av_w, gl_w], axis=1)
    b24 = jnp.concatenate([au_b, jnp.zeros((H,), jnp.float32), gl_b])
    chaos = (jax.random.normal(jax.random.key(42), (N, H), dtype=jnp.float32)
             * chaos_factor)
    z32 = jnp.zeros((_STRIPE, 32), jnp.float32)
    z128 = jnp.zeros((_STRIPE, DIM), jnp.float32)

    xfc, tsl, tsv = _project(x, fc_w, fc_b, w24, b24, chaos)
    accp, pb = _sc_pass1(tsl, tsv, src1d, dst1d, z32)
    tde = _energy(accp, tsl)
    msgp = _sc_pass2(tde, xfc, pb, src1d, dst1d, z128)
    return _ffn(xfc, msgp[0], msgp[1], ff1_w, ff1_b, ff2_w, ff2_b)


# speedup vs baseline: 38.5067x; 1.2501x over previous
import functools

import jax
import jax.numpy as jnp
from jax import lax
from jax.experimental import pallas as pl
from jax.experimental.pallas import tpu as pltpu
from jax.experimental.pallas import tpu_sc as plsc

N = 10000
E = 320000
DIM = 128
H = 8
HID = 2 * DIM

_BLK = 1000
_NC, _NS = 2, 16
_CHUNK = 40
_CPT = E // (_NC * _NS * _CHUNK)
_NPAD = 10240
_STRIPE = _NPAD // _NS


def _proj_body(x_ref, fcw_ref, fcb_ref, w24_ref, b24_ref, chaos_ref,
               xfc_ref, tsl_ref, tsv_ref):
    xfc = jnp.dot(x_ref[...], fcw_ref[...],
                  preferred_element_type=jnp.float32) + fcb_ref[...]
    xfc_ref[...] = xfc
    hv = jnp.dot(xfc, w24_ref[...],
                 preferred_element_type=jnp.float32) + b24_ref[...]
    su = hv[:, 0:8]
    sv = hv[:, 8:16]
    lap = hv[:, 16:24] + chaos_ref[...]
    zeros = jnp.zeros_like(su)
    tsl_ref[...] = jnp.concatenate([su, su, zeros, lap], axis=1)
    tsv_ref[...] = jnp.concatenate([sv, sv], axis=1)


def _project(x, fc_w, fc_b, w24, b24, chaos):
    return pl.pallas_call(
        _proj_body,
        grid=(N // _BLK,),
        in_specs=[
            pl.BlockSpec((_BLK, DIM), lambda i: (i, 0)),
            pl.BlockSpec((DIM, DIM), lambda i: (0, 0)),
            pl.BlockSpec((DIM,), lambda i: (0,)),
            pl.BlockSpec((DIM, 24), lambda i: (0, 0)),
            pl.BlockSpec((24,), lambda i: (0,)),
            pl.BlockSpec((_BLK, H), lambda i: (i, 0)),
        ],
        out_specs=[
            pl.BlockSpec((_BLK, DIM), lambda i: (i, 0)),
            pl.BlockSpec((_BLK, 32), lambda i: (i, 0)),
            pl.BlockSpec((_BLK, 16), lambda i: (i, 0)),
        ],
        out_shape=[
            jax.ShapeDtypeStruct((N, DIM), jnp.float32),
            jax.ShapeDtypeStruct((N, 32), jnp.float32),
            jax.ShapeDtypeStruct((N, 16), jnp.float32),
        ],
    )(x, fc_w, fc_b, w24, b24, chaos)


def _sc_pass1(tsl, tsv, src3d, dst3d, z32):
    mesh = plsc.VectorSubcoreMesh(core_axis_name="c", subcore_axis_name="s")

    @functools.partial(
        pl.kernel,
        mesh=mesh,
        compiler_params=pltpu.CompilerParams(use_tc_tiling_on_sc=False),
        out_type=[
            jax.ShapeDtypeStruct((_NC, _NPAD, 32), jnp.float32),
            jax.ShapeDtypeStruct((E, 16), jnp.float32),
        ],
        scratch_types=[
            pltpu.VMEM((_CPT, _CHUNK), jnp.int32),
            pltpu.VMEM((_CPT, _CHUNK), jnp.int32),
            pltpu.VMEM((_CHUNK, 32), jnp.float32),
            pltpu.VMEM((_CHUNK, 32), jnp.float32),
            pltpu.VMEM((_CHUNK, 16), jnp.float32),
            pltpu.VMEM((_CHUNK, 16), jnp.float32),
            pltpu.VMEM((_CHUNK, 32), jnp.float32),
            pltpu.VMEM((_CHUNK, 32), jnp.float32),
            pltpu.VMEM((_CHUNK, 16), jnp.float32),
            pltpu.VMEM((_CHUNK, 16), jnp.float32),
            pltpu.VMEM_SHARED((_NPAD, 32), jnp.float32),
            pltpu.SemaphoreType.DMA,
            pltpu.SemaphoreType.DMA,
            pltpu.SemaphoreType.DMA,
            pltpu.SemaphoreType.DMA,
            pltpu.SemaphoreType.DMA,
            pltpu.SemaphoreType.DMA,
        ],
    )
    def kern(tsl_hbm, tsv_hbm, src_hbm, dst_hbm, z32_hbm, accp_hbm, pb_hbm,
             sidx_all, didx_all, tslg0, tslg1, tsvg0, tsvg1, sb0, sb1,
             pbuf0, pbuf1, acc_sh, gsem0, gsem1, osem0, osem1, psem0, psem1):
        c = lax.axis_index("c")
        s = lax.axis_index("s")
        tile = c * _NS + s
        tslg = (tslg0, tslg1)
        tsvg = (tsvg0, tsvg1)
        sb = (sb0, sb1)
        pbuf = (pbuf0, pbuf1)
        gsem = (gsem0, gsem1)
        osem = (osem0, osem1)
        psem = (psem0, psem1)

        pltpu.sync_copy(src_hbm.at[tile], sidx_all)
        pltpu.sync_copy(dst_hbm.at[tile], didx_all)
        pltpu.sync_copy(z32_hbm, acc_sh.at[pl.ds(s * _STRIPE, _STRIPE)])
        lane = lax.iota(jnp.int32, 16)
        one0 = jnp.where(lane < 1, 1.0, 0.0).astype(jnp.float32)
        mask8 = lane < 8

        @pl.loop(0, _CHUNK)
        def _(r):
            sb0[r, pl.ds(16, 16)] = one0
            sb1[r, pl.ds(16, 16)] = one0

        plsc.subcore_barrier()

        def issue_in(jj, b):
            h1 = pltpu.async_copy(tsl_hbm.at[sidx_all.at[jj]], tslg[b],
                                  gsem[b])
            h2 = pltpu.async_copy(tsv_hbm.at[didx_all.at[jj]], tsvg[b],
                                  gsem[b])
            return (h1, h2)

        def issue_out(jj, b):
            row = tile * _CPT + jj
            h1 = pltpu.async_copy(sb[b], acc_sh.at[didx_all.at[jj]], osem[b],
                                  add=True)
            h2 = pltpu.async_copy(pbuf[b],
                                  pb_hbm.at[pl.ds(row * _CHUNK, _CHUNK)],
                                  psem[b])
            return (h1, h2)

        def compute(b):
            @pl.loop(0, _CHUNK)
            def _(r):
                a0 = tslg[b][r, pl.ds(0, 16)]
                a1 = tslg[b][r, pl.ds(16, 16)]
                b0 = tsvg[b][r, pl.ds(0, 16)]
                t = a0 + b0
                t = jnp.maximum(t, 0.2 * t)
                p = jnp.exp(t)
                sb[b][r, pl.ds(0, 16)] = jnp.where(mask8, p, a1)
                pbuf[b][r, pl.ds(0, 16)] = p

        @pl.loop(0, _CPT, step=2)
        def _(j):
            hins = [issue_in(j, 0), issue_in(j + 1, 1)]
            houts = []
            for b in range(2):
                for h in hins[b]:
                    h.wait()
                compute(b)
                houts.append(issue_out(j + b, b))
            for hs in houts:
                for h in hs:
                    h.wait()

        plsc.subcore_barrier()
        pltpu.sync_copy(acc_sh.at[pl.ds(s * _STRIPE, _STRIPE)],
                        accp_hbm.at[c].at[pl.ds(s * _STRIPE, _STRIPE)])

    return kern(tsl, tsv, src3d, dst3d, z32)


def _energy_body(a0_ref, a1_ref, tsl_ref, tde_ref):
    acc = a0_ref[...] + a1_ref[...]
    denom = acc[:, 0:8]
    nbr = acc[:, 8:16]
    indeg = acc[:, 16:17]
    lap = tsl_ref[:, 24:32]
    energy = lap - nbr / jnp.maximum(indeg, 1.0)
    tde_ref[...] = jnp.concatenate([denom, denom, energy, energy], axis=1)


def _energy(accp, tsl):
    return pl.pallas_call(
        _energy_body,
        grid=(N // _BLK,),
        in_specs=[
            pl.BlockSpec((_BLK, 32), lambda i: (i, 0)),
            pl.BlockSpec((_BLK, 32), lambda i: (i, 0)),
            pl.BlockSpec((_BLK, 32), lambda i: (i, 0)),
        ],
        out_specs=pl.BlockSpec((_BLK, 32), lambda i: (i, 0)),
        out_shape=jax.ShapeDtypeStruct((_NPAD, 32), jnp.float32),
    )(accp[0], accp[1], tsl)


def _sc_pass2(tde, xfc, pb, src3d, dst3d, z128):
    mesh = plsc.VectorSubcoreMesh(core_axis_name="c", subcore_axis_name="s")

    @functools.partial(
        pl.kernel,
        mesh=mesh,
        compiler_params=pltpu.CompilerParams(use_tc_tiling_on_sc=False),
        out_type=jax.ShapeDtypeStruct((_NC, _NPAD, DIM), jnp.float32),
        scratch_types=[
            pltpu.VMEM((_CPT, _CHUNK), jnp.int32),
            pltpu.VMEM((_CPT, _CHUNK), jnp.int32),
            pltpu.VMEM((_CHUNK, 32), jnp.float32),
            pltpu.VMEM((_CHUNK, 32), jnp.float32),
            pltpu.VMEM((_CHUNK, 32), jnp.float32),
            pltpu.VMEM((_CHUNK, 32), jnp.float32),
            pltpu.VMEM((_CHUNK, DIM), jnp.float32),
            pltpu.VMEM((_CHUNK, DIM), jnp.float32),
            pltpu.VMEM((_CHUNK, DIM), jnp.float32),
            pltpu.VMEM((_CHUNK, DIM), jnp.float32),
            pltpu.VMEM((_CHUNK, 16), jnp.float32),
            pltpu.VMEM((_CHUNK, 16), jnp.float32),
            pltpu.VMEM_SHARED((_NPAD, DIM), jnp.float32),
            pltpu.SemaphoreType.DMA,
            pltpu.SemaphoreType.DMA,
            pltpu.SemaphoreType.DMA,
            pltpu.SemaphoreType.DMA,
            pltpu.SemaphoreType.DMA,
            pltpu.SemaphoreType.DMA,
        ],
    )
    def kern(tde_hbm, xfc_hbm, pb_hbm, src_hbm, dst_hbm, z128_hbm, msgp_hbm,
             didx_all, sidx_all, gd0, gd1, gs0, gs1, xb0, xb1, yb0, yb1,
             pbuf0, pbuf1, msg_sh, gsem0, gsem1, osem0, osem1,
             psem0, psem1):
        c = lax.axis_index("c")
        s = lax.axis_index("s")
        tile = c * _NS + s
        gd = (gd0, gd1)
        gs = (gs0, gs1)
        xb = (xb0, xb1)
        yb = (yb0, yb1)
        pbuf = (pbuf0, pbuf1)
        gsem = (gsem0, gsem1)
        osem = (osem0, osem1)
        psem = (psem0, psem1)

        pltpu.sync_copy(dst_hbm.at[tile], didx_all)
        pltpu.sync_copy(src_hbm.at[tile], sidx_all)
        pltpu.sync_copy(z128_hbm, msg_sh.at[pl.ds(s * _STRIPE, _STRIPE)])
        plsc.subcore_barrier()

        def issue_in(jj, b):
            row = tile * _CPT + jj
            h1 = pltpu.async_copy(tde_hbm.at[didx_all.at[jj]], gd[b], gsem[b])
            h2 = pltpu.async_copy(tde_hbm.at[sidx_all.at[jj]], gs[b], gsem[b])
            h3 = pltpu.async_copy(xfc_hbm.at[sidx_all.at[jj]], xb[b], gsem[b])
            h4 = pltpu.async_copy(pb_hbm.at[pl.ds(row * _CHUNK, _CHUNK)],
                                  pbuf[b], psem[b])
            return (h1, h2, h3, h4)

        def issue_out(jj, b):
            h = pltpu.async_copy(yb[b], msg_sh.at[didx_all.at[jj]], osem[b],
                                 add=True)
            return (h,)

        def compute(b):
            @pl.loop(0, _CHUNK)
            def _(r):
                z = gd[b][r, pl.ds(16, 16)] - gs[b][r, pl.ds(16, 16)]
                sig = 1.0 / (1.0 + jnp.exp(-z))
                q = pbuf[b][r, pl.ds(0, 16)] / gd[b][r, pl.ds(0, 16)]
                m = q * sig
                for kk in range(DIM // 16):
                    yb[b][r, pl.ds(kk * 16, 16)] = (
                        xb[b][r, pl.ds(kk * 16, 16)] * m)

        @pl.loop(0, _CPT, step=2)
        def _(j):
            hins = [issue_in(j, 0), issue_in(j + 1, 1)]
            houts = []
            for b in range(2):
                for h in hins[b]:
                    h.wait()
                compute(b)
                houts.append(issue_out(j + b, b))
            for hs in houts:
                for h in hs:
                    h.wait()

        plsc.subcore_barrier()
        pltpu.sync_copy(msg_sh.at[pl.ds(s * _STRIPE, _STRIPE)],
                        msgp_hbm.at[c].at[pl.ds(s * _STRIPE, _STRIPE)])

    return kern(tde, xfc, pb, src3d, dst3d, z128)


def _ffn_body(xfc_ref, m0_ref, m1_ref, w1a_ref, w1b_ref, b1_ref, w2_ref,
              b2_ref, out_ref):
    msg = m0_ref[...] + m1_ref[...]
    h = (jnp.dot(xfc_ref[...], w1a_ref[...], preferred_element_type=jnp.float32)
         + jnp.dot(msg, w1b_ref[...], preferred_element_type=jnp.float32)
         + b1_ref[...])
    h = 0.5 * h * (1.0 + jax.lax.erf(h * 0.7071067811865476))
    out_ref[...] = jnp.dot(h, w2_ref[...],
                           preferred_element_type=jnp.float32) + b2_ref[...]


def _ffn(xfc, m0, m1, ff1_w, ff1_b, ff2_w, ff2_b):
    return pl.pallas_call(
        _ffn_body,
        grid=(N // _BLK,),
        in_specs=[
            pl.BlockSpec((_BLK, DIM), lambda i: (i, 0)),
            pl.BlockSpec((_BLK, DIM), lambda i: (i, 0)),
            pl.BlockSpec((_BLK, DIM), lambda i: (i, 0)),
            pl.BlockSpec((DIM, HID), lambda i: (0, 0)),
            pl.BlockSpec((DIM, HID), lambda i: (0, 0)),
            pl.BlockSpec((HID,), lambda i: (0,)),
            pl.BlockSpec((HID, DIM), lambda i: (0, 0)),
            pl.BlockSpec((DIM,), lambda i: (0,)),
        ],
        out_specs=pl.BlockSpec((_BLK, DIM), lambda i: (i, 0)),
        out_shape=jax.ShapeDtypeStruct((N, DIM), jnp.float32),
    )(xfc, m0, m1, ff1_w[:DIM], ff1_w[DIM:], ff1_b, ff2_w, ff2_b)


def kernel(x, edge_index, fc_w, fc_b, au_w, au_b, av_w, gl_w, gl_b,
           chaos_factor, ff1_w, ff1_b, ff2_w, ff2_b):
    src3d = edge_index[0].reshape(_NC * _NS, _CPT, _CHUNK)
    dst3d = edge_index[1].reshape(_NC * _NS, _CPT, _CHUNK)

    w24 = jnp.concatenate([au_w, av_w, gl_w], axis=1)
    b24 = jnp.concatenate([au_b, jnp.zeros((H,), jnp.float32), gl_b])
    chaos = (jax.random.normal(jax.random.key(42), (N, H), dtype=jnp.float32)
             * chaos_factor)
    z32 = jnp.zeros((_STRIPE, 32), jnp.float32)
    z128 = jnp.zeros((_STRIPE, DIM), jnp.float32)

    xfc, tsl, tsv = _project(x, fc_w, fc_b, w24, b24, chaos)
    accp, pb = _sc_pass1(tsl, tsv, src3d, dst3d, z32)
    tde = _energy(accp, tsl)
    msgp = _sc_pass2(tde, xfc, pb, src3d, dst3d, z128)
    return _ffn(xfc, msgp[0], msgp[1], ff1_w, ff1_b, ff2_w, ff2_b)
